# Initial kernel scaffold; baseline (speedup 1.0000x reference)
import jax
import jax.numpy as jnp
from jax.experimental import pallas as pl


def kernel(x, gate_w, w1, b1, w2, b2):
    raise NotImplementedError("write your pallas kernel here")


# speedup vs baseline: 1.1304x; 1.1304x over previous
import functools

import jax
import jax.numpy as jnp
from jax.experimental import pallas as pl
from jax.experimental.pallas import tpu as pltpu

N = 1576
D = 384
DFF = 1536
E = 6
TEMP = 0.07
EPS = 1e-6

NPAD = 1600
BN = 400
NB = NPAD // BN
EPADG = 8


def _moe_body(x_ref, gwn_ref, w1_ref, b1_ref, w2_ref, b2_ref, out_ref,
              comb_ref):
    j = pl.program_id(1)

    @pl.when(j == 0)
    def _router():
        xb = x_ref[...]
        nrm = jnp.sqrt(jnp.sum(xb * xb, axis=1, keepdims=True))
        xn = xb / (nrm + EPS)
        gw = gwn_ref[...]
        gn = gw / (jnp.sqrt(jnp.sum(gw * gw, axis=1, keepdims=True)) + EPS)
        logits = jnp.dot(xn, gn.T,
                         preferred_element_type=jnp.float32) / TEMP
        cols = jax.lax.broadcasted_iota(jnp.int32, (BN, EPADG), 1)
        logits = jnp.where(cols < E, logits, -1e30)
        m1 = jnp.max(logits, axis=1, keepdims=True)
        i1 = jnp.min(jnp.where(logits == m1, cols, EPADG), axis=1,
                     keepdims=True)
        masked = jnp.where(cols == i1, -1e30, logits)
        m2 = jnp.max(masked, axis=1, keepdims=True)
        i2 = jnp.min(jnp.where(masked == m2, cols, EPADG), axis=1,
                     keepdims=True)
        g1 = 1.0 / (1.0 + jnp.exp(m2 - m1))
        g2 = 1.0 - g1
        comb_ref[...] = (g1 * (cols == i1).astype(jnp.float32)
                         + g2 * (cols == i2).astype(jnp.float32))

    xb16 = x_ref[...].astype(jnp.bfloat16)
    h = jnp.dot(xb16, w1_ref[0], preferred_element_type=jnp.float32)
    h = jax.nn.gelu(h + b1_ref[0])
    y = jnp.dot(h.astype(jnp.bfloat16), w2_ref[0],
                preferred_element_type=jnp.float32)
    y = y + b2_ref[0]
    allcols = jax.lax.broadcasted_iota(jnp.int32, (BN, EPADG), 1)
    cb = jnp.sum(jnp.where(allcols == j, comb_ref[...], 0.0), axis=1,
                 keepdims=True)
    contrib = cb * y

    @pl.when(j == 0)
    def _init():
        out_ref[...] = contrib

    @pl.when(j > 0)
    def _acc():
        out_ref[...] += contrib


@jax.jit
def kernel(x, gate_w, w1, b1, w2, b2):
    xp = jnp.pad(x, ((0, NPAD - N), (0, 0)))
    gwp = jnp.pad(gate_w, ((0, EPADG - E), (0, 0)))
    w1b = w1.astype(jnp.bfloat16)
    w2b = w2.astype(jnp.bfloat16)

    out = pl.pallas_call(
        _moe_body,
        grid=(NB, E),
        in_specs=[
            pl.BlockSpec((BN, D), lambda i, j: (i, 0)),
            pl.BlockSpec((EPADG, D), lambda i, j: (0, 0)),
            pl.BlockSpec((1, D, DFF), lambda i, j: (j, 0, 0)),
            pl.BlockSpec((1, 1, DFF), lambda i, j: (j, 0, 0)),
            pl.BlockSpec((1, DFF, D), lambda i, j: (j, 0, 0)),
            pl.BlockSpec((1, 1, D), lambda i, j: (j, 0, 0)),
        ],
        out_specs=pl.BlockSpec((BN, D), lambda i, j: (i, 0)),
        out_shape=jax.ShapeDtypeStruct((NPAD, D), jnp.float32),
        scratch_shapes=[pltpu.VMEM((BN, EPADG), jnp.float32)],
        compiler_params=pltpu.CompilerParams(
            dimension_semantics=("arbitrary", "arbitrary"),
        ),
    )(xp, gwp, w1b, b1[:, None, :], w2b, b2[:, None, :])
    return out[:N]


# speedup vs baseline: 1.5046x; 1.3311x over previous
import jax
import jax.numpy as jnp
from jax.experimental import pallas as pl
from jax.experimental.pallas import tpu as pltpu

N = 1576
D = 384
DFF = 1536
E = 6
TEMP = 0.07
EPS = 1e-6

NPAD = 1600
EPADG = 8


def _moe_body(x_ref, gwn_ref, w1_ref, b1_ref, w2_ref, b2_ref, out_ref,
              comb_ref):
    j = pl.program_id(0)

    @pl.when(j == 0)
    def _router():
        xb = x_ref[...]
        nrm = jnp.sqrt(jnp.sum(xb * xb, axis=1, keepdims=True))
        xn = xb / (nrm + EPS)
        gw = gwn_ref[...]
        gn = gw / (jnp.sqrt(jnp.sum(gw * gw, axis=1, keepdims=True)) + EPS)
        logits = jnp.dot(xn, gn.T,
                         preferred_element_type=jnp.float32) / TEMP
        cols = jax.lax.broadcasted_iota(jnp.int32, (NPAD, EPADG), 1)
        logits = jnp.where(cols < E, logits, -1e30)
        m1 = jnp.max(logits, axis=1, keepdims=True)
        i1 = jnp.min(jnp.where(logits == m1, cols, EPADG), axis=1,
                     keepdims=True)
        masked = jnp.where(cols == i1, -1e30, logits)
        m2 = jnp.max(masked, axis=1, keepdims=True)
        i2 = jnp.min(jnp.where(masked == m2, cols, EPADG), axis=1,
                     keepdims=True)
        g1 = 1.0 / (1.0 + jnp.exp(m2 - m1))
        g2 = 1.0 - g1
        comb_ref[...] = (g1 * (cols == i1).astype(jnp.float32)
                         + g2 * (cols == i2).astype(jnp.float32))

    xb16 = x_ref[...].astype(jnp.bfloat16)
    h = jnp.dot(xb16, w1_ref[0].astype(jnp.bfloat16),
                preferred_element_type=jnp.float32)
    h = jax.nn.gelu(h + b1_ref[0])
    y = jnp.dot(h.astype(jnp.bfloat16), w2_ref[0].astype(jnp.bfloat16),
                preferred_element_type=jnp.float32)
    y = y + b2_ref[0]
    allcols = jax.lax.broadcasted_iota(jnp.int32, (NPAD, EPADG), 1)
    cb = jnp.sum(jnp.where(allcols == j, comb_ref[...], 0.0), axis=1,
                 keepdims=True)
    contrib = cb * y

    @pl.when(j == 0)
    def _init():
        out_ref[...] = contrib

    @pl.when(j > 0)
    def _acc():
        out_ref[...] += contrib


@jax.jit
def kernel(x, gate_w, w1, b1, w2, b2):
    xp = jnp.pad(x, ((0, NPAD - N), (0, 0)))
    gwp = jnp.pad(gate_w, ((0, EPADG - E), (0, 0)))

    out = pl.pallas_call(
        _moe_body,
        grid=(E,),
        in_specs=[
            pl.BlockSpec((NPAD, D), lambda j: (0, 0)),
            pl.BlockSpec((EPADG, D), lambda j: (0, 0)),
            pl.BlockSpec((1, D, DFF), lambda j: (j, 0, 0)),
            pl.BlockSpec((1, 1, DFF), lambda j: (j, 0, 0)),
            pl.BlockSpec((1, DFF, D), lambda j: (j, 0, 0)),
            pl.BlockSpec((1, 1, D), lambda j: (j, 0, 0)),
        ],
        out_specs=pl.BlockSpec((NPAD, D), lambda j: (0, 0)),
        out_shape=jax.ShapeDtypeStruct((NPAD, D), jnp.float32),
        scratch_shapes=[pltpu.VMEM((NPAD, EPADG), jnp.float32)],
        compiler_params=pltpu.CompilerParams(
            dimension_semantics=("arbitrary",),
        ),
    )(xp, gwp, w1, b1[:, None, :], w2, b2[:, None, :])
    return out[:N]


# speedup vs baseline: 1.6734x; 1.1122x over previous
import jax
import jax.numpy as jnp
from jax.experimental import pallas as pl
from jax.experimental.pallas import tpu as pltpu

N = 1576
D = 384
DFF = 1536
E = 6
TEMP = 0.07
EPS = 1e-6

NPAD = 1600
EPADG = 8


def _moe_body(x_ref, gwn_ref, w1_ref, b1_ref, w2_ref, b2_ref, out_ref,
              comb_ref):
    j = pl.program_id(0)

    @pl.when(j == 0)
    def _router():
        xb = x_ref[...]
        nrm = jnp.sqrt(jnp.sum(xb * xb, axis=1, keepdims=True))
        xn = xb / (nrm + EPS)
        gw = gwn_ref[...]
        gn = gw / (jnp.sqrt(jnp.sum(gw * gw, axis=1, keepdims=True)) + EPS)
        logits = jnp.dot(xn, gn.T,
                         preferred_element_type=jnp.float32) / TEMP
        cols = jax.lax.broadcasted_iota(jnp.int32, (NPAD, EPADG), 1)
        logits = jnp.where(cols < E, logits, -1e30)
        m1 = jnp.max(logits, axis=1, keepdims=True)
        i1 = jnp.min(jnp.where(logits == m1, cols, EPADG), axis=1,
                     keepdims=True)
        masked = jnp.where(cols == i1, -1e30, logits)
        m2 = jnp.max(masked, axis=1, keepdims=True)
        i2 = jnp.min(jnp.where(masked == m2, cols, EPADG), axis=1,
                     keepdims=True)
        g1 = 1.0 / (1.0 + jnp.exp(m2 - m1))
        g2 = 1.0 - g1
        comb_ref[...] = (g1 * (cols == i1).astype(jnp.float32)
                         + g2 * (cols == i2).astype(jnp.float32))

    xb16 = x_ref[...].astype(jnp.bfloat16)
    h = jnp.dot(xb16, w1_ref[0].astype(jnp.bfloat16),
                preferred_element_type=jnp.float32)
    h = jax.nn.gelu(h.astype(jnp.bfloat16)
                    + b1_ref[0].astype(jnp.bfloat16))
    y = jnp.dot(h, w2_ref[0].astype(jnp.bfloat16),
                preferred_element_type=jnp.float32)
    y = y + b2_ref[0]
    allcols = jax.lax.broadcasted_iota(jnp.int32, (NPAD, EPADG), 1)
    cb = jnp.sum(jnp.where(allcols == j, comb_ref[...], 0.0), axis=1,
                 keepdims=True)
    contrib = cb * y

    @pl.when(j == 0)
    def _init():
        out_ref[...] = contrib

    @pl.when(j > 0)
    def _acc():
        out_ref[...] += contrib


@jax.jit
def kernel(x, gate_w, w1, b1, w2, b2):
    xp = jnp.pad(x, ((0, NPAD - N), (0, 0)))
    gwp = jnp.pad(gate_w, ((0, EPADG - E), (0, 0)))

    out = pl.pallas_call(
        _moe_body,
        grid=(E,),
        in_specs=[
            pl.BlockSpec((NPAD, D), lambda j: (0, 0)),
            pl.BlockSpec((EPADG, D), lambda j: (0, 0)),
            pl.BlockSpec((1, D, DFF), lambda j: (j, 0, 0)),
            pl.BlockSpec((1, 1, DFF), lambda j: (j, 0, 0)),
            pl.BlockSpec((1, DFF, D), lambda j: (j, 0, 0)),
            pl.BlockSpec((1, 1, D), lambda j: (j, 0, 0)),
        ],
        out_specs=pl.BlockSpec((NPAD, D), lambda j: (0, 0)),
        out_shape=jax.ShapeDtypeStruct((NPAD, D), jnp.float32),
        scratch_shapes=[pltpu.VMEM((NPAD, EPADG), jnp.float32)],
        compiler_params=pltpu.CompilerParams(
            dimension_semantics=("arbitrary",),
        ),
    )(xp, gwp, w1, b1[:, None, :], w2, b2[:, None, :])
    return out[:N]
